# Initial kernel scaffold; baseline (speedup 1.0000x reference)
import jax
import jax.numpy as jnp
from jax.experimental import pallas as pl


def kernel(x, edge_index, W1, b1, W2, b2):
    raise NotImplementedError("write your pallas kernel here")


# speedup vs baseline: 16.4340x; 16.4340x over previous
import jax
import jax.numpy as jnp
from jax import lax
from jax.experimental import pallas as pl
from jax.experimental.pallas import tpu as pltpu
from jax.experimental.pallas import tpu_sc as plsc

N_NODES = 10000
N_EDGES = 320000
NFEAT = 128
NHID = 256
NCLASS = 40

NC = 2
NS = 16
NW = NC * NS
HHALF = NHID // NC

_E_PER_W = N_EDGES // NW
_HIST = 2 * N_NODES


def _deg_body(src_hbm, dst_hbm, out_hbm, sbuf, dbuf, hist):
    c = lax.axis_index("c")
    s = lax.axis_index("s")
    wid = s * NC + c
    zeros16 = jnp.zeros((16,), jnp.float32)

    def zb(t, carry):
        hist[pl.ds(t * 16, 16)] = zeros16
        return carry

    lax.fori_loop(0, _HIST // 16, zb, 0)

    base = wid * _E_PER_W
    pltpu.sync_copy(src_hbm.at[pl.ds(base, _E_PER_W)], sbuf)
    pltpu.sync_copy(dst_hbm.at[pl.ds(base, _E_PER_W)], dbuf)

    ones16 = jnp.ones((16,), jnp.float32)

    def body(j, carry):
        isrc = sbuf[pl.ds(j * 16, 16)]
        plsc.addupdate_scatter(hist, [isrc], ones16)
        idst = dbuf[pl.ds(j * 16, 16)] + N_NODES
        plsc.addupdate_scatter(hist, [idst], ones16)
        return carry

    lax.fori_loop(0, _E_PER_W // 16, body, 0)
    pltpu.sync_copy(hist, out_hbm.at[pl.ds(wid * _HIST, _HIST)])


_deg_kernel = pl.kernel(
    _deg_body,
    out_type=jax.ShapeDtypeStruct((NW * _HIST,), jnp.float32),
    mesh=plsc.VectorSubcoreMesh(core_axis_name="c", subcore_axis_name="s"),
    compiler_params=pltpu.CompilerParams(needs_layout_passes=False),
    scratch_types=[
        pltpu.VMEM((_E_PER_W,), jnp.int32),
        pltpu.VMEM((_E_PER_W,), jnp.int32),
        pltpu.VMEM((_HIST,), jnp.float32),
    ],
)


def _norm_body(hist_ref, nout_ref, nin_ref):
    deg = jnp.sum(hist_ref[...], axis=0)
    nrm = lax.rsqrt(jnp.maximum(deg, 1.0))
    nout_ref[...] = nrm[:N_NODES].reshape(N_NODES, 1)
    nin_ref[...] = nrm[N_NODES:].reshape(N_NODES, 1)


def _norm_call(hist):
    return pl.pallas_call(
        _norm_body,
        out_shape=(
            jax.ShapeDtypeStruct((N_NODES, 1), jnp.float32),
            jax.ShapeDtypeStruct((N_NODES, 1), jnp.float32),
        ),
    )(hist)


_RB = 400
_NB = N_NODES // _RB


def _mm1_body(x_ref, w1_ref, nrm_ref, out_ref):
    out_ref[...] = (
        jnp.dot(x_ref[...], w1_ref[...], preferred_element_type=jnp.float32)
        * nrm_ref[...]
    )


def _mm1_call(x, W1, nout):
    return pl.pallas_call(
        _mm1_body,
        grid=(NC, _NB),
        in_specs=[
            pl.BlockSpec((_RB, NFEAT), lambda h, i: (i, 0)),
            pl.BlockSpec((NFEAT, HHALF), lambda h, i: (0, h)),
            pl.BlockSpec((_RB, 1), lambda h, i: (i, 0)),
        ],
        out_specs=pl.BlockSpec((_RB, HHALF), lambda h, i: (h * _NB + i, 0)),
        out_shape=jax.ShapeDtypeStruct((NC * N_NODES, HHALF), jnp.float32),
    )(x, W1, nout)


_E_PER_S = N_EDGES // NS
_CHUNK = 80
_NCH = _E_PER_S // _CHUNK
_CPR = 632
_CLAST = N_NODES - (NS - 1) * _CPR
_ZCH = 80


def _agg_body(hs_hbm, src_hbm, dst_hbm, out_hbm,
              gidx0, gidx1, didx0, didx1, rows0, rows1, zrow, agg_sp,
              gsem0, gsem1, ssem0, ssem1):
    c = lax.axis_index("c")
    s = lax.axis_index("s")
    coff = c * N_NODES

    zeros16 = jnp.zeros((16,), jnp.float32)

    def zb(t, carry):
        r = t // (HHALF // 16)
        col = t % (HHALF // 16)
        zrow[r, pl.ds(col * 16, 16)] = zeros16
        return carry

    lax.fori_loop(0, _ZCH * (HHALF // 16), zb, 0)

    def zero_range(start, nfull, tail):
        for j in range(nfull):
            pltpu.sync_copy(zrow, agg_sp.at[pl.ds(start + j * _ZCH, _ZCH)])
        if tail:
            pltpu.sync_copy(
                zrow.at[pl.ds(0, tail)],
                agg_sp.at[pl.ds(start + nfull * _ZCH, tail)],
            )

    @pl.when(s < NS - 1)
    def _():
        zero_range(s * _CPR, 7, 72)

    @pl.when(s == NS - 1)
    def _():
        zero_range((NS - 1) * _CPR, 6, 40)

    plsc.subcore_barrier()

    rows = (rows0, rows1)
    gidx = (gidx0, gidx1)
    didx = (didx0, didx1)
    gsem = (gsem0, gsem1)
    ssem = (ssem0, ssem1)

    def prep_and_fire(j, b):
        base = s * _E_PER_S + j * _CHUNK
        pltpu.sync_copy(src_hbm.at[pl.ds(base, _CHUNK)], gidx[b])
        pltpu.sync_copy(dst_hbm.at[pl.ds(base, _CHUNK)], didx[b])
        for q in range(_CHUNK // 16):
            gidx[b][pl.ds(q * 16, 16)] = gidx[b][pl.ds(q * 16, 16)] + coff
        pltpu.async_copy(hs_hbm.at[gidx[b]], rows[b], gsem[b])

    def wait_gather(b):
        pltpu.make_async_copy(hs_hbm.at[gidx[b]], rows[b], gsem[b]).wait()

    def fire_scatter(b):
        pltpu.async_copy(rows[b], agg_sp.at[didx[b]], ssem[b], add=True)

    def wait_scatter(b):
        pltpu.make_async_copy(rows[b], agg_sp.at[didx[b]], ssem[b]).wait()

    prep_and_fire(0, 0)

    def pair(i, carry):
        for b in range(2):
            k = 2 * i + b
            nb = 1 - b

            @pl.when(k + 1 < _NCH)
            def _():
                @pl.when(k >= 1)
                def _():
                    wait_scatter(nb)
                prep_and_fire(k + 1, nb)

            wait_gather(b)
            fire_scatter(b)
        return carry

    lax.fori_loop(0, _NCH // 2, pair, 0)
    wait_scatter(0)
    wait_scatter(1)
    plsc.subcore_barrier()

    @pl.when(s < NS - 1)
    def _():
        pltpu.sync_copy(
            agg_sp.at[pl.ds(s * _CPR, _CPR)],
            out_hbm.at[pl.ds(c * N_NODES + s * _CPR, _CPR)],
        )

    @pl.when(s == NS - 1)
    def _():
        pltpu.sync_copy(
            agg_sp.at[pl.ds((NS - 1) * _CPR, _CLAST)],
            out_hbm.at[pl.ds(c * N_NODES + (NS - 1) * _CPR, _CLAST)],
        )


_agg_kernel = pl.kernel(
    _agg_body,
    out_type=jax.ShapeDtypeStruct((NC * N_NODES, HHALF), jnp.float32),
    mesh=plsc.VectorSubcoreMesh(core_axis_name="c", subcore_axis_name="s"),
    compiler_params=pltpu.CompilerParams(needs_layout_passes=False),
    scratch_types=[
        pltpu.VMEM((_CHUNK,), jnp.int32),
        pltpu.VMEM((_CHUNK,), jnp.int32),
        pltpu.VMEM((_CHUNK,), jnp.int32),
        pltpu.VMEM((_CHUNK,), jnp.int32),
        pltpu.VMEM((_CHUNK, HHALF), jnp.float32),
        pltpu.VMEM((_CHUNK, HHALF), jnp.float32),
        pltpu.VMEM((_ZCH, HHALF), jnp.float32),
        pltpu.VMEM_SHARED((N_NODES, HHALF), jnp.float32),
        pltpu.SemaphoreType.DMA,
        pltpu.SemaphoreType.DMA,
        pltpu.SemaphoreType.DMA,
        pltpu.SemaphoreType.DMA,
    ],
)


def _out_body(aggA_ref, aggB_ref, nin_ref, b1_ref, w2_ref, b2_ref, out_ref):
    agg = jnp.concatenate([aggA_ref[...], aggB_ref[...]], axis=1)
    h = agg * nin_ref[...] + b1_ref[...]
    h = jnp.maximum(h, 0.0)
    out_ref[...] = (
        jnp.dot(h, w2_ref[...], preferred_element_type=jnp.float32) + b2_ref[...]
    )


def _out_call(aggst, nin, b1, W2, b2):
    return pl.pallas_call(
        _out_body,
        grid=(_NB,),
        in_specs=[
            pl.BlockSpec((_RB, HHALF), lambda i: (i, 0)),
            pl.BlockSpec((_RB, HHALF), lambda i: (_NB + i, 0)),
            pl.BlockSpec((_RB, 1), lambda i: (i, 0)),
            pl.BlockSpec((1, NHID), lambda i: (0, 0)),
            pl.BlockSpec((NHID, NCLASS), lambda i: (0, 0)),
            pl.BlockSpec((1, NCLASS), lambda i: (0, 0)),
        ],
        out_specs=pl.BlockSpec((_RB, NCLASS), lambda i: (i, 0)),
        out_shape=jax.ShapeDtypeStruct((N_NODES, NCLASS), jnp.float32),
    )(aggst, aggst, nin, b1, W2, b2)


def kernel(x, edge_index, W1, b1, W2, b2):
    src = edge_index[0].astype(jnp.int32)
    dst = edge_index[1].astype(jnp.int32)
    histflat = _deg_kernel(src, dst)
    hist = histflat.reshape(NW, _HIST)
    nout, nin = _norm_call(hist)
    hs = _mm1_call(x, W1, nout)
    aggst = _agg_kernel(hs, src, dst)
    out = _out_call(aggst, nin, b1.reshape(1, NHID), W2, b2.reshape(1, NCLASS))
    return out


# speedup vs baseline: 21.7552x; 1.3238x over previous
import jax
import jax.numpy as jnp
from jax import lax
from jax.experimental import pallas as pl
from jax.experimental.pallas import tpu as pltpu
from jax.experimental.pallas import tpu_sc as plsc

N_NODES = 10000
N_EDGES = 320000
NFEAT = 128
NHID = 256
NCLASS = 40

NC = 2
NS = 16
NW = NC * NS
HHALF = NHID // NC

_E_PER_W = N_EDGES // NW
_HIST = 2 * N_NODES


def _deg_body(src_hbm, dst_hbm, out_hbm, sbuf, dbuf, hist):
    c = lax.axis_index("c")
    s = lax.axis_index("s")
    wid = s * NC + c
    zeros16 = jnp.zeros((16,), jnp.float32)

    def zb(t, carry):
        hist[pl.ds(t * 16, 16)] = zeros16
        return carry

    lax.fori_loop(0, _HIST // 16, zb, 0)

    base = wid * _E_PER_W
    pltpu.sync_copy(src_hbm.at[pl.ds(base, _E_PER_W)], sbuf)
    pltpu.sync_copy(dst_hbm.at[pl.ds(base, _E_PER_W)], dbuf)

    ones16 = jnp.ones((16,), jnp.float32)

    def body(j, carry):
        isrc = sbuf[pl.ds(j * 16, 16)]
        plsc.addupdate_scatter(hist, [isrc], ones16)
        idst = dbuf[pl.ds(j * 16, 16)] + N_NODES
        plsc.addupdate_scatter(hist, [idst], ones16)
        return carry

    lax.fori_loop(0, _E_PER_W // 16, body, 0)
    pltpu.sync_copy(hist, out_hbm.at[pl.ds(wid * _HIST, _HIST)])


_deg_kernel = pl.kernel(
    _deg_body,
    out_type=jax.ShapeDtypeStruct((NW * _HIST,), jnp.float32),
    mesh=plsc.VectorSubcoreMesh(core_axis_name="c", subcore_axis_name="s"),
    compiler_params=pltpu.CompilerParams(needs_layout_passes=False),
    scratch_types=[
        pltpu.VMEM((_E_PER_W,), jnp.int32),
        pltpu.VMEM((_E_PER_W,), jnp.int32),
        pltpu.VMEM((_HIST,), jnp.float32),
    ],
)


def _norm_body(hist_ref, nout_ref, nin_ref):
    deg = jnp.sum(hist_ref[...], axis=0)
    nrm = lax.rsqrt(jnp.maximum(deg, 1.0))
    nout_ref[...] = nrm[:N_NODES].reshape(N_NODES, 1)
    nin_ref[...] = nrm[N_NODES:].reshape(N_NODES, 1)


def _norm_call(hist):
    return pl.pallas_call(
        _norm_body,
        out_shape=(
            jax.ShapeDtypeStruct((N_NODES, 1), jnp.float32),
            jax.ShapeDtypeStruct((N_NODES, 1), jnp.float32),
        ),
    )(hist)


_RB = 400
_NB = N_NODES // _RB


def _mm1_body(x_ref, w1_ref, nrm_ref, out_ref):
    out_ref[...] = (
        jnp.dot(x_ref[...], w1_ref[...], preferred_element_type=jnp.float32)
        * nrm_ref[...]
    )


def _mm1_call(x, W1, nout):
    return pl.pallas_call(
        _mm1_body,
        grid=(NC, _NB),
        in_specs=[
            pl.BlockSpec((_RB, NFEAT), lambda h, i: (i, 0)),
            pl.BlockSpec((NFEAT, HHALF), lambda h, i: (0, h)),
            pl.BlockSpec((_RB, 1), lambda h, i: (i, 0)),
        ],
        out_specs=pl.BlockSpec((_RB, HHALF), lambda h, i: (h * _NB + i, 0)),
        out_shape=jax.ShapeDtypeStruct((NC * N_NODES, HHALF), jnp.float32),
    )(x, W1, nout)


_E_PER_S = N_EDGES // NS
_CHUNK = 80
_NCH = _E_PER_S // _CHUNK
_NBLK = 5
_BCH = _NCH // _NBLK
_CPR = 632
_CLAST = N_NODES - (NS - 1) * _CPR
_ZCH = 80


def _agg_body(hs_hbm, src_hbm, dst_hbm, out_hbm,
              gblk, dblk, rows0, rows1, zrow, agg_sp,
              gsem0, gsem1, ssem0, ssem1):
    c = lax.axis_index("c")
    s = lax.axis_index("s")
    coff = c * N_NODES

    zeros16 = jnp.zeros((16,), jnp.float32)

    def zb(t, carry):
        r = t // (HHALF // 16)
        col = t % (HHALF // 16)
        zrow[r, pl.ds(col * 16, 16)] = zeros16
        return carry

    lax.fori_loop(0, _ZCH * (HHALF // 16), zb, 0)

    def zero_range(start, nfull, tail):
        for j in range(nfull):
            pltpu.sync_copy(zrow, agg_sp.at[pl.ds(start + j * _ZCH, _ZCH)])
        if tail:
            pltpu.sync_copy(
                zrow.at[pl.ds(0, tail)],
                agg_sp.at[pl.ds(start + nfull * _ZCH, tail)],
            )

    @pl.when(s < NS - 1)
    def _():
        zero_range(s * _CPR, 7, 72)

    @pl.when(s == NS - 1)
    def _():
        zero_range((NS - 1) * _CPR, 6, 40)

    plsc.subcore_barrier()

    rows = (rows0, rows1)
    gsem = (gsem0, gsem1)
    ssem = (ssem0, ssem1)

    def fire_gather(j, b):
        pltpu.async_copy(hs_hbm.at[gblk.at[j]], rows[b], gsem[b])

    def wait_gather(b):
        pltpu.make_async_copy(hs_hbm.at[gblk.at[0]], rows[b], gsem[b]).wait()

    def fire_scatter(k, b):
        pltpu.async_copy(rows[b], agg_sp.at[dblk.at[k]], ssem[b], add=True)

    def wait_scatter(b):
        pltpu.make_async_copy(rows[b], agg_sp.at[dblk.at[0]], ssem[b]).wait()

    for blk in range(_NBLK):
        pltpu.sync_copy(src_hbm.at[s, blk], gblk)
        pltpu.sync_copy(dst_hbm.at[s, blk], dblk)

        def gx(t, carry):
            r = t // (_CHUNK // 16)
            q = t % (_CHUNK // 16)
            gblk[r, pl.ds(q * 16, 16)] = gblk[r, pl.ds(q * 16, 16)] + coff
            return carry

        lax.fori_loop(0, _BCH * (_CHUNK // 16), gx, 0)

        fire_gather(0, 0)

        def pair(i, carry):
            for b in range(2):
                k = 2 * i + b
                nb = 1 - b

                @pl.when(k + 1 < _BCH)
                def _():
                    @pl.when(k >= 1)
                    def _():
                        wait_scatter(nb)
                    fire_gather(k + 1, nb)

                wait_gather(b)
                fire_scatter(k, b)
            return carry

        lax.fori_loop(0, _BCH // 2, pair, 0)
        wait_scatter(0)
        wait_scatter(1)

    plsc.subcore_barrier()

    @pl.when(s < NS - 1)
    def _():
        pltpu.sync_copy(
            agg_sp.at[pl.ds(s * _CPR, _CPR)],
            out_hbm.at[pl.ds(c * N_NODES + s * _CPR, _CPR)],
        )

    @pl.when(s == NS - 1)
    def _():
        pltpu.sync_copy(
            agg_sp.at[pl.ds((NS - 1) * _CPR, _CLAST)],
            out_hbm.at[pl.ds(c * N_NODES + (NS - 1) * _CPR, _CLAST)],
        )


_agg_kernel = pl.kernel(
    _agg_body,
    out_type=jax.ShapeDtypeStruct((NC * N_NODES, HHALF), jnp.float32),
    mesh=plsc.VectorSubcoreMesh(core_axis_name="c", subcore_axis_name="s"),
    compiler_params=pltpu.CompilerParams(needs_layout_passes=False),
    scratch_types=[
        pltpu.VMEM((_BCH, _CHUNK), jnp.int32),
        pltpu.VMEM((_BCH, _CHUNK), jnp.int32),
        pltpu.VMEM((_CHUNK, HHALF), jnp.float32),
        pltpu.VMEM((_CHUNK, HHALF), jnp.float32),
        pltpu.VMEM((_ZCH, HHALF), jnp.float32),
        pltpu.VMEM_SHARED((N_NODES, HHALF), jnp.float32),
        pltpu.SemaphoreType.DMA,
        pltpu.SemaphoreType.DMA,
        pltpu.SemaphoreType.DMA,
        pltpu.SemaphoreType.DMA,
    ],
)


def _out_body(aggA_ref, aggB_ref, nin_ref, b1_ref, w2_ref, b2_ref, out_ref):
    agg = jnp.concatenate([aggA_ref[...], aggB_ref[...]], axis=1)
    h = agg * nin_ref[...] + b1_ref[...]
    h = jnp.maximum(h, 0.0)
    out_ref[...] = (
        jnp.dot(h, w2_ref[...], preferred_element_type=jnp.float32) + b2_ref[...]
    )


def _out_call(aggst, nin, b1, W2, b2):
    return pl.pallas_call(
        _out_body,
        grid=(_NB,),
        in_specs=[
            pl.BlockSpec((_RB, HHALF), lambda i: (i, 0)),
            pl.BlockSpec((_RB, HHALF), lambda i: (_NB + i, 0)),
            pl.BlockSpec((_RB, 1), lambda i: (i, 0)),
            pl.BlockSpec((1, NHID), lambda i: (0, 0)),
            pl.BlockSpec((NHID, NCLASS), lambda i: (0, 0)),
            pl.BlockSpec((1, NCLASS), lambda i: (0, 0)),
        ],
        out_specs=pl.BlockSpec((_RB, NCLASS), lambda i: (i, 0)),
        out_shape=jax.ShapeDtypeStruct((N_NODES, NCLASS), jnp.float32),
    )(aggst, aggst, nin, b1, W2, b2)


def kernel(x, edge_index, W1, b1, W2, b2):
    src = edge_index[0].astype(jnp.int32)
    dst = edge_index[1].astype(jnp.int32)
    histflat = _deg_kernel(src, dst)
    hist = histflat.reshape(NW, _HIST)
    nout, nin = _norm_call(hist)
    hs = _mm1_call(x, W1, nout)
    src4 = src.reshape(NS, _NBLK, _BCH, _CHUNK)
    dst4 = dst.reshape(NS, _NBLK, _BCH, _CHUNK)
    aggst = _agg_kernel(hs, src4, dst4)
    out = _out_call(aggst, nin, b1.reshape(1, NHID), W2, b2.reshape(1, NCLASS))
    return out
